# Initial kernel scaffold; baseline (speedup 1.0000x reference)
import jax
import jax.numpy as jnp
from jax.experimental import pallas as pl


def kernel(x, norm_GraphSAGE, W, b):
    raise NotImplementedError("write your pallas kernel here")


# speedup vs baseline: 1.0286x; 1.0286x over previous
import jax
import jax.numpy as jnp
from jax.experimental import pallas as pl


def _fused_sage_kernel(a_ref, x_ref, xblk_ref, w_ref, b_ref, out_ref):
    f = x_ref.shape[1]
    agg = jnp.dot(a_ref[...], x_ref[...], preferred_element_type=jnp.float32)
    out = jnp.dot(xblk_ref[...], w_ref[:f, :], preferred_element_type=jnp.float32)
    out += jnp.dot(agg, w_ref[f:, :], preferred_element_type=jnp.float32)
    out += b_ref[...]
    out_ref[...] = jnp.maximum(out, 0.0)


def kernel(x, norm_GraphSAGE, W, b):
    n, f = x.shape
    f_out = W.shape[1]
    bm = 400
    assert n % bm == 0
    b2 = b.reshape(1, f_out)
    return pl.pallas_call(
        _fused_sage_kernel,
        grid=(n // bm,),
        in_specs=[
            pl.BlockSpec((bm, n), lambda i: (i, 0)),
            pl.BlockSpec((n, f), lambda i: (0, 0)),
            pl.BlockSpec((bm, f), lambda i: (i, 0)),
            pl.BlockSpec(W.shape, lambda i: (0, 0)),
            pl.BlockSpec((1, f_out), lambda i: (0, 0)),
        ],
        out_specs=pl.BlockSpec((bm, f_out), lambda i: (i, 0)),
        out_shape=jax.ShapeDtypeStruct((n, f_out), jnp.float32),
    )(norm_GraphSAGE, x, x, W, b2)
